# baseline (speedup 1.0000x reference)
import functools

import jax
import jax.numpy as jnp
from jax import lax
from jax.experimental import pallas as pl
from jax.experimental.pallas import tpu as pltpu
from jax.experimental.pallas import tpu_sc as plsc

_RANK = 64
_NC = 2
_NS = 16
_NW = _NC * _NS


def _sc_gather(ent, rel, x0, x1, x2):
    B = x0.shape[0]
    b_per_w = B // _NW
    mesh = plsc.VectorSubcoreMesh(core_axis_name="c", subcore_axis_name="s",
                                  num_cores=_NC)
    row = jax.ShapeDtypeStruct((B, _RANK), jnp.float32)

    @functools.partial(
        pl.kernel,
        mesh=mesh,
        out_type=(row, row, row),
        compiler_params=pltpu.CompilerParams(use_tc_tiling_on_sc=False),
        scratch_types=[
            pltpu.VMEM((b_per_w,), jnp.int32),
            pltpu.VMEM((b_per_w,), jnp.int32),
            pltpu.VMEM((b_per_w,), jnp.int32),
            pltpu.VMEM((b_per_w, _RANK), jnp.float32),
            pltpu.VMEM((b_per_w, _RANK), jnp.float32),
            pltpu.VMEM((b_per_w, _RANK), jnp.float32),
            pltpu.SemaphoreType.DMA,
        ],
    )
    def gather_kernel(ent_hbm, rel_hbm, x0_hbm, x1_hbm, x2_hbm,
                      lhs_hbm, r_hbm, rhs_hbm,
                      i0, i1, i2, lv, rv, hv, sem):
        wid = lax.axis_index("s") * _NC + lax.axis_index("c")
        base = wid * b_per_w
        pltpu.sync_copy(x0_hbm.at[pl.ds(base, b_per_w)], i0)
        pltpu.sync_copy(x1_hbm.at[pl.ds(base, b_per_w)], i1)
        pltpu.sync_copy(x2_hbm.at[pl.ds(base, b_per_w)], i2)
        c0 = pltpu.async_copy(ent_hbm.at[i0], lv, sem)
        c1 = pltpu.async_copy(rel_hbm.at[i1], rv, sem)
        c2 = pltpu.async_copy(ent_hbm.at[i2], hv, sem)
        c0.wait()
        c1.wait()
        c2.wait()
        pltpu.sync_copy(lv, lhs_hbm.at[pl.ds(base, b_per_w)])
        pltpu.sync_copy(rv, r_hbm.at[pl.ds(base, b_per_w)])
        pltpu.sync_copy(hv, rhs_hbm.at[pl.ds(base, b_per_w)])

    return gather_kernel(ent, rel, x0, x1, x2)


def _tc_scores(lhs, r, ent, eblk=2048):
    B = lhs.shape[0]
    n_ent = ent.shape[0]

    def mm_kernel(lhs_ref, r_ref, ent_ref, out_ref):
        q = lhs_ref[...] * r_ref[...]
        out_ref[...] = lax.dot_general(
            q, ent_ref[...], (((1,), (1,)), ((), ())),
            preferred_element_type=jnp.float32)

    return pl.pallas_call(
        mm_kernel,
        grid=(pl.cdiv(n_ent, eblk),),
        in_specs=[
            pl.BlockSpec((B, _RANK), lambda j: (0, 0)),
            pl.BlockSpec((B, _RANK), lambda j: (0, 0)),
            pl.BlockSpec((eblk, _RANK), lambda j: (j, 0)),
        ],
        out_specs=pl.BlockSpec((B, eblk), lambda j: (0, j)),
        out_shape=jax.ShapeDtypeStruct((B, n_ent), jnp.float32),
    )(lhs, r, ent)


@jax.jit
def kernel(x, ent, rel):
    x0 = x[:, 0]
    x1 = x[:, 1]
    x2 = x[:, 2]
    lhs, r, rhs = _sc_gather(ent, rel, x0, x1, x2)
    scores = _tc_scores(lhs, r, ent)
    return (scores, (lhs, r, rhs))


# speedup vs baseline: 1.0039x; 1.0039x over previous
import functools

import jax
import jax.numpy as jnp
from jax import lax
from jax.experimental import pallas as pl
from jax.experimental.pallas import tpu as pltpu
from jax.experimental.pallas import tpu_sc as plsc

_RANK = 64
_NC = 2
_NS = 16
_NW = _NC * _NS


def _sc_gather(ent, rel, x0, x1, x2):
    B = x0.shape[0]
    b_per_w = B // _NW
    mesh = plsc.VectorSubcoreMesh(core_axis_name="c", subcore_axis_name="s",
                                  num_cores=_NC)
    row = jax.ShapeDtypeStruct((B, _RANK), jnp.float32)

    @functools.partial(
        pl.kernel,
        mesh=mesh,
        out_type=(row, row, row),
        compiler_params=pltpu.CompilerParams(use_tc_tiling_on_sc=False),
        scratch_types=[
            pltpu.VMEM((b_per_w,), jnp.int32),
            pltpu.VMEM((b_per_w,), jnp.int32),
            pltpu.VMEM((b_per_w,), jnp.int32),
            pltpu.VMEM((b_per_w, _RANK), jnp.float32),
            pltpu.VMEM((b_per_w, _RANK), jnp.float32),
            pltpu.VMEM((b_per_w, _RANK), jnp.float32),
            pltpu.SemaphoreType.DMA,
        ],
    )
    def gather_kernel(ent_hbm, rel_hbm, x0_hbm, x1_hbm, x2_hbm,
                      lhs_hbm, r_hbm, rhs_hbm,
                      i0, i1, i2, lv, rv, hv, sem):
        wid = lax.axis_index("s") * _NC + lax.axis_index("c")
        base = wid * b_per_w
        pltpu.sync_copy(x0_hbm.at[pl.ds(base, b_per_w)], i0)
        pltpu.sync_copy(x1_hbm.at[pl.ds(base, b_per_w)], i1)
        pltpu.sync_copy(x2_hbm.at[pl.ds(base, b_per_w)], i2)
        c0 = pltpu.async_copy(ent_hbm.at[i0], lv, sem)
        c1 = pltpu.async_copy(rel_hbm.at[i1], rv, sem)
        c2 = pltpu.async_copy(ent_hbm.at[i2], hv, sem)
        c0.wait()
        c1.wait()
        c2.wait()
        pltpu.sync_copy(lv, lhs_hbm.at[pl.ds(base, b_per_w)])
        pltpu.sync_copy(rv, r_hbm.at[pl.ds(base, b_per_w)])
        pltpu.sync_copy(hv, rhs_hbm.at[pl.ds(base, b_per_w)])

    return gather_kernel(ent, rel, x0, x1, x2)


def _tc_scores(lhs, r, ent, eblk=4096):
    B = lhs.shape[0]
    n_ent = ent.shape[0]

    def mm_kernel(lhs_ref, r_ref, ent_ref, out_ref):
        q = lhs_ref[...] * r_ref[...]
        out_ref[...] = lax.dot_general(
            q, ent_ref[...], (((1,), (1,)), ((), ())),
            preferred_element_type=jnp.float32)

    return pl.pallas_call(
        mm_kernel,
        grid=(pl.cdiv(n_ent, eblk),),
        in_specs=[
            pl.BlockSpec((B, _RANK), lambda j: (0, 0)),
            pl.BlockSpec((B, _RANK), lambda j: (0, 0)),
            pl.BlockSpec((eblk, _RANK), lambda j: (j, 0)),
        ],
        out_specs=pl.BlockSpec((B, eblk), lambda j: (0, j)),
        out_shape=jax.ShapeDtypeStruct((B, n_ent), jnp.float32),
    )(lhs, r, ent)


@jax.jit
def kernel(x, ent, rel):
    x0 = x[:, 0]
    x1 = x[:, 1]
    x2 = x[:, 2]
    lhs, r, rhs = _sc_gather(ent, rel, x0, x1, x2)
    scores = _tc_scores(lhs, r, ent)
    return (scores, (lhs, r, rhs))


# speedup vs baseline: 1.0557x; 1.0516x over previous
import functools

import jax
import jax.numpy as jnp
from jax import lax
from jax.experimental import pallas as pl
from jax.experimental.pallas import tpu as pltpu
from jax.experimental.pallas import tpu_sc as plsc

_RANK = 64
_NC = 2
_NS = 16
_NW = _NC * _NS


def _sc_gather(ent, rel, x0, x1, x2):
    B = x0.shape[0]
    b_per_w = B // _NW
    mesh = plsc.VectorSubcoreMesh(core_axis_name="c", subcore_axis_name="s",
                                  num_cores=_NC)
    row = jax.ShapeDtypeStruct((B, _RANK), jnp.float32)

    @functools.partial(
        pl.kernel,
        mesh=mesh,
        out_type=(row, row, row),
        compiler_params=pltpu.CompilerParams(use_tc_tiling_on_sc=False),
        scratch_types=[
            pltpu.VMEM((b_per_w,), jnp.int32),
            pltpu.VMEM((b_per_w,), jnp.int32),
            pltpu.VMEM((b_per_w,), jnp.int32),
            pltpu.VMEM((b_per_w, _RANK), jnp.float32),
            pltpu.VMEM((b_per_w, _RANK), jnp.float32),
            pltpu.VMEM((b_per_w, _RANK), jnp.float32),
            pltpu.SemaphoreType.DMA,
        ],
    )
    def gather_kernel(ent_hbm, rel_hbm, x0_hbm, x1_hbm, x2_hbm,
                      lhs_hbm, r_hbm, rhs_hbm,
                      i0, i1, i2, lv, rv, hv, sem):
        wid = lax.axis_index("s") * _NC + lax.axis_index("c")
        base = wid * b_per_w
        pltpu.sync_copy(x0_hbm.at[pl.ds(base, b_per_w)], i0)
        pltpu.sync_copy(x1_hbm.at[pl.ds(base, b_per_w)], i1)
        pltpu.sync_copy(x2_hbm.at[pl.ds(base, b_per_w)], i2)
        c0 = pltpu.async_copy(ent_hbm.at[i0], lv, sem)
        c1 = pltpu.async_copy(rel_hbm.at[i1], rv, sem)
        c2 = pltpu.async_copy(ent_hbm.at[i2], hv, sem)
        c0.wait()
        c1.wait()
        c2.wait()
        pltpu.sync_copy(lv, lhs_hbm.at[pl.ds(base, b_per_w)])
        pltpu.sync_copy(rv, r_hbm.at[pl.ds(base, b_per_w)])
        pltpu.sync_copy(hv, rhs_hbm.at[pl.ds(base, b_per_w)])

    return gather_kernel(ent, rel, x0, x1, x2)


def _tc_scores(lhs, r, ent, eblk=4096):
    B = lhs.shape[0]
    n_ent = ent.shape[0]

    def mm_kernel(lhs_ref, r_ref, ent_ref, out_ref):
        q = lhs_ref[...] * r_ref[...]
        out_ref[...] = lax.dot_general(
            q, ent_ref[...], (((1,), (1,)), ((), ())),
            preferred_element_type=jnp.float32)

    return pl.pallas_call(
        mm_kernel,
        grid=(pl.cdiv(n_ent, eblk),),
        in_specs=[
            pl.BlockSpec((B, _RANK), lambda j: (0, 0)),
            pl.BlockSpec((B, _RANK), lambda j: (0, 0)),
            pl.BlockSpec((eblk, _RANK), lambda j: (j, 0)),
        ],
        out_specs=pl.BlockSpec((B, eblk), lambda j: (0, j)),
        out_shape=jax.ShapeDtypeStruct((B, n_ent), jnp.float32),
    )(lhs, r, ent)


@jax.jit
def kernel(x, ent, rel):
    x0 = x[:, 0]
    x1 = x[:, 1]
    x2 = x[:, 2]
    lhs, r, rhs = jnp.take(ent, x0, axis=0), jnp.take(rel, x1, axis=0), jnp.take(ent, x2, axis=0)
    scores = _tc_scores(lhs, r, ent)
    return (scores, (lhs, r, rhs))
